# baseline (speedup 1.0000x reference)
import functools

import jax
import jax.numpy as jnp
from jax import lax
from jax.experimental import pallas as pl
from jax.experimental.pallas import tpu as pltpu
from jax.experimental.pallas import tpu_sc as plsc

N = 10000
E = 160000
IN = 128
H = 32
EF = 4
G = 64
L = 4
M_ST = 3

NW = 32
CH = 128
EP = 163840
CPW = EP // NW // CH
NPAD = 10240
RPT = NPAD // 16
BE = 512


def _h0_body(x_ref, wf_ref, bf_ref, h_ref):
    h_ref[...] = (
        jnp.dot(x_ref[...], wf_ref[...], preferred_element_type=jnp.float32)
        + bf_ref[...]
    )


def _u_body(ea_ref, w1_ref, b1_ref, u_ref):
    v = (
        jnp.dot(ea_ref[...], w1_ref[...], preferred_element_type=jnp.float32)
        + b1_ref[...]
    )
    u_ref[...] = v * jax.nn.sigmoid(v)


def _msg_body(u_ref, xj_ref, w2_ref, b2_ref, o_ref):
    w = (
        jnp.dot(u_ref[...], w2_ref[...], preferred_element_type=jnp.float32)
        + b2_ref[...]
    )
    xj = xj_ref[...]
    msg = xj[:, 0:1] * w[:, 0:H]
    for i in range(1, H):
        msg = msg + xj[:, i:i + 1] * w[:, H * i:H * i + H]
    o_ref[...] = msg


def _upd_body(h_ref, p0_ref, p1_ref, wr_ref, bc_ref, o_ref):
    h = h_ref[...]
    aggr = p0_ref[...] + p1_ref[...]
    hw = jnp.dot(h, wr_ref[...], preferred_element_type=jnp.float32)
    o_ref[...] = h + ((aggr + hw) + bc_ref[...])


def _s2s_body(h_ref, b_ref, wih_ref, bih_ref, whh_ref, bhh_ref,
              wo1_ref, bo1_ref, wo2_ref, bo2_ref, o_ref):
    h = h_ref[...]
    bat = b_ref[...]
    seg = lax.broadcasted_iota(jnp.int32, (N, G), 1)
    ohb = bat == seg
    oh = ohb.astype(jnp.float32)
    q_star = jnp.zeros((G, 2 * H), jnp.float32)
    hx = jnp.zeros((G, H), jnp.float32)
    cx = jnp.zeros((G, H), jnp.float32)
    for _ in range(M_ST):
        gates = (
            jnp.dot(q_star, wih_ref[...], preferred_element_type=jnp.float32)
            + bih_ref[...]
            + jnp.dot(hx, whh_ref[...], preferred_element_type=jnp.float32)
            + bhh_ref[...]
        )
        i_g = jax.nn.sigmoid(gates[:, 0:H])
        f_g = jax.nn.sigmoid(gates[:, H:2 * H])
        g_g = jnp.tanh(gates[:, 2 * H:3 * H])
        o_g = jax.nn.sigmoid(gates[:, 3 * H:4 * H])
        cx = f_g * cx + i_g * g_g
        hx = o_g * jnp.tanh(cx)
        q = hx
        qb = jnp.dot(oh, q, preferred_element_type=jnp.float32,
                     precision=lax.Precision.HIGHEST)
        e = jnp.sum(h * qb, axis=1, keepdims=True)
        em = jnp.max(jnp.where(ohb, e, -3e38), axis=0, keepdims=True)
        e_exp = jnp.exp(e - jnp.sum(oh * em, axis=1, keepdims=True))
        den = jnp.sum(oh * e_exp, axis=0, keepdims=True)
        a = e_exp / jnp.sum(oh * den, axis=1, keepdims=True)
        r = lax.dot_general(oh, a * h, (((0,), (0,)), ((), ())),
                            preferred_element_type=jnp.float32,
                            precision=lax.Precision.HIGHEST)
        q_star = jnp.concatenate([q, r], axis=1)
    o = (
        jnp.dot(q_star, wo1_ref[...], preferred_element_type=jnp.float32)
        + bo1_ref[...]
    )
    o = o * jax.nn.sigmoid(o)
    o_ref[...] = (
        jnp.dot(o, wo2_ref[...], preferred_element_type=jnp.float32)
        + bo2_ref[...]
    )


def _full(*shape):
    return pl.BlockSpec(shape, lambda *_: tuple(0 for _ in shape))


_h0_call = pl.pallas_call(
    _h0_body,
    grid=(1,),
    in_specs=[_full(N, IN), _full(IN, H), _full(1, H)],
    out_specs=_full(N, H),
    out_shape=jax.ShapeDtypeStruct((N, H), jnp.float32),
)

_BU = 4096
_u_call = pl.pallas_call(
    _u_body,
    grid=(EP // _BU,),
    in_specs=[pl.BlockSpec((_BU, EF), lambda i: (i, 0)),
              _full(EF, 2 * H), _full(1, 2 * H)],
    out_specs=pl.BlockSpec((_BU, 2 * H), lambda i: (i, 0)),
    out_shape=jax.ShapeDtypeStruct((EP, 2 * H), jnp.float32),
)

_msg_call = pl.pallas_call(
    _msg_body,
    grid=(EP // BE,),
    in_specs=[
        pl.BlockSpec((BE, 2 * H), lambda i: (i, 0)),
        pl.BlockSpec((BE, H), lambda i: (i, 0)),
        _full(2 * H, H * H), _full(1, H * H),
    ],
    out_specs=pl.BlockSpec((BE, H), lambda i: (i, 0)),
    out_shape=jax.ShapeDtypeStruct((EP, H), jnp.float32),
)

_upd_call = pl.pallas_call(
    _upd_body,
    grid=(1,),
    in_specs=[_full(N, H), _full(N, H), _full(N, H), _full(H, H), _full(1, H)],
    out_specs=_full(N, H),
    out_shape=jax.ShapeDtypeStruct((N, H), jnp.float32),
)

_s2s_call = pl.pallas_call(
    _s2s_body,
    grid=(1,),
    in_specs=[_full(N, H), _full(N, 1),
              _full(2 * H, 4 * H), _full(1, 4 * H),
              _full(H, 4 * H), _full(1, 4 * H),
              _full(2 * H, H), _full(1, H),
              _full(H, 1), _full(1, 1)],
    out_specs=_full(G, 1),
    out_shape=jax.ShapeDtypeStruct((G, 1), jnp.float32),
)


@functools.cache
def _sc_kernels():
    mesh = plsc.VectorSubcoreMesh(core_axis_name="c", subcore_axis_name="s")

    @functools.partial(
        pl.kernel, mesh=mesh,
        compiler_params=pltpu.CompilerParams(use_tc_tiling_on_sc=False),
        out_type=jax.ShapeDtypeStruct((EP, H), jnp.float32),
        scratch_types=[
            pltpu.VMEM((CPW, CH), jnp.int32),
            pltpu.VMEM((CH, H), jnp.float32),
            pltpu.SemaphoreType.DMA,
        ],
    )
    def sc_gather(h_hbm, src_hbm, xj_hbm, idx_v, rows_v, sem):
        wid = lax.axis_index("s") * 2 + lax.axis_index("c")
        base = wid * CPW
        pltpu.sync_copy(src_hbm.at[pl.ds(base, CPW)], idx_v)

        def body(j, carry):
            pltpu.async_copy(h_hbm.at[idx_v.at[j]], rows_v, sem).wait()
            pltpu.sync_copy(rows_v, xj_hbm.at[pl.ds((base + j) * CH, CH)])
            return carry

        lax.fori_loop(0, CPW, body, 0)

    @functools.partial(
        pl.kernel, mesh=mesh,
        compiler_params=pltpu.CompilerParams(use_tc_tiling_on_sc=False),
        out_type=jax.ShapeDtypeStruct((2, NPAD * H), jnp.float32),
        scratch_types=[
            pltpu.VMEM((CH + 16,), jnp.int32),
            pltpu.VMEM((CH * H,), jnp.float32),
            pltpu.VMEM((H,), jnp.float32),
            pltpu.VMEM((48,), jnp.int32),
            pltpu.VMEM_SHARED((NPAD * H,), jnp.float32),
        ],
    )
    def sc_scatter(msg_hbm, dst_hbm, cuts_hbm, z_hbm, out_hbm,
                   idx_v, buf_v, row_v, cuts_v, acc_sh):
        c = lax.axis_index("c")
        s = lax.axis_index("s")
        wid = c * 16 + s
        pltpu.sync_copy(z_hbm, acc_sh.at[pl.ds(s * RPT * H, RPT * H)])
        pltpu.sync_copy(cuts_hbm, cuts_v.at[pl.ds(0, 40)])
        plsc.subcore_barrier()
        cvec = cuts_v[pl.ds(wid, 16)]
        cw = cvec[0]
        cw1 = cvec[1]
        c0 = lax.div(cw, CH)
        c1 = lax.div(cw1 + (CH - 1), CH)
        trash = jnp.int32(NPAD - 1)

        def flush(cur, a0, a1):
            row_v[pl.ds(0, 16)] = a0
            row_v[pl.ds(16, 16)] = a1
            pltpu.sync_copy(row_v, acc_sh.at[pl.ds(cur * H, H)])

        def edge_body(e, carry):
            cur, a0, a1, cc = carry
            gidx = cc * CH + e
            d_raw = idx_v[pl.ds(e, 16)][0]
            inside = jnp.logical_and(gidx >= cw, gidx < cw1)
            d = jnp.where(inside, d_raw, trash)
            r0 = buf_v[pl.ds(e * H, 16)]
            r1 = buf_v[pl.ds(e * H + 16, 16)]
            same = d == cur
            lax.cond(same, lambda: None, lambda: flush(cur, a0, a1))
            sv = jnp.where(same, jnp.float32(1.0), jnp.float32(0.0))
            a0 = a0 * sv + r0
            a1 = a1 * sv + r1
            return d, a0, a1, cc

        def chunk_body(cc, carry):
            cur, a0, a1, _ = carry
            pltpu.sync_copy(dst_hbm.at[cc], idx_v.at[pl.ds(0, CH)])
            pltpu.sync_copy(msg_hbm.at[pl.ds(cc * CH * H, CH * H)], buf_v)
            return lax.fori_loop(0, CH, edge_body, (cur, a0, a1, cc))

        zero16 = jnp.zeros((16,), jnp.float32)
        cur, a0, a1, _ = lax.fori_loop(
            c0, c1, chunk_body, (trash, zero16, zero16, c0))
        flush(cur, a0, a1)
        plsc.subcore_barrier()
        pltpu.sync_copy(acc_sh.at[pl.ds(s * RPT * H, RPT * H)],
                        out_hbm.at[c, pl.ds(s * RPT * H, RPT * H)])

    return sc_gather, sc_scatter


def kernel(x, edge_index, edge_attr, batch, W_first, b_first, W_nn1, b_nn1,
           W_nn2, b_nn2, W_root, b_conv, W_ih, b_ih, W_hh, b_hh,
           W_o1, b_o1, W_o2, b_o2):
    src = edge_index[0].astype(jnp.int32)
    dst = edge_index[1].astype(jnp.int32)
    perm = jnp.argsort(dst, stable=True)
    src = jnp.take(src, perm)
    dst = jnp.take(dst, perm)
    edge_attr = jnp.take(edge_attr, perm, axis=0)
    src_p = jnp.concatenate(
        [src, jnp.zeros((EP - E,), jnp.int32)]).reshape(EP // CH, CH)
    dst_flat = jnp.concatenate([dst, jnp.full((EP - E,), N, jnp.int32)])
    dst_p = dst_flat.reshape(EP // CH, CH)
    targets = dst_flat[jnp.arange(NW) * (EP // NW)]
    cuts = jnp.concatenate([
        jnp.searchsorted(dst_flat, targets, side='left').astype(jnp.int32),
        jnp.full((8,), EP, jnp.int32)])
    ea_p = jnp.concatenate(
        [edge_attr, jnp.zeros((EP - E, EF), jnp.float32)], axis=0)
    bat2d = batch.astype(jnp.int32).reshape(N, 1)
    zeros_sc = jnp.zeros((RPT * H,), jnp.float32)

    sc_gather, sc_scatter = _sc_kernels()
    h = _h0_call(x, W_first, b_first.reshape(1, H))
    u = _u_call(ea_p, W_nn1, b_nn1.reshape(1, 2 * H))
    for l in range(L):
        xj = sc_gather(h, src_p)
        msg = _msg_call(u, xj, W_nn2, b_nn2.reshape(1, H * H))
        p = sc_scatter(msg.reshape(EP * H), dst_p, cuts, zeros_sc)
        p = p.reshape(2, NPAD, H)
        h = _upd_call(h, p[0, :N], p[1, :N], W_root[l], b_conv[l].reshape(1, H))
    out = _s2s_call(h, bat2d, W_ih, b_ih.reshape(1, 4 * H),
                    W_hh, b_hh.reshape(1, 4 * H), W_o1, b_o1.reshape(1, H),
                    W_o2, b_o2.reshape(1, 1))
    return out.reshape(G)
